# baseline (speedup 1.0000x reference)
import functools

import jax
import jax.numpy as jnp
from jax import lax
from jax.experimental import pallas as pl
from jax.experimental.pallas import tpu as pltpu
from jax.experimental.pallas import tpu_sc as plsc

NSC = 2
TPS = 16
NT = NSC * TPS
K = 128
NRING = 4
LANES = 16


@functools.partial(jax.jit, static_argnums=(3, 4, 5))
def _sc_aggregate(x, pk, pw, N, D, NCH):
    rpt = 8 * (-(-N // (TPS * 8)))
    NP = TPS * rpt
    assert NCH % NRING == 0 and NCH >= NRING

    mesh = plsc.VectorSubcoreMesh(core_axis_name="c", subcore_axis_name="s")

    @functools.partial(
        pl.kernel,
        mesh=mesh,
        out_type=jax.ShapeDtypeStruct((NSC, NP, D), jnp.float32),
        scratch_types=[
            pltpu.VMEM((NRING, 2, K), jnp.int32),
            pltpu.VMEM((NRING, K), jnp.float32),
            pltpu.VMEM((K, D), jnp.float32),
            pltpu.VMEM((K, D), jnp.float32),
            pltpu.VMEM_SHARED((NP, D), jnp.float32),
        ]
        + [pltpu.SemaphoreType.DMA for _ in range(NRING + 4)],
    )
    def sc(x_hbm, pk_hbm, pw_hbm, out_hbm, idxr, ewr, buf0, buf1, acc,
           *sems):
        si = sems[:NRING]
        sg = sems[NRING:NRING + 2]
        ss = sems[NRING + 2:NRING + 4]
        bufs = (buf0, buf1)
        cid = lax.axis_index("c")
        sid = lax.axis_index("s")
        tid = cid * TPS + sid

        def zrow(r, carry):
            for c in range(D // LANES):
                buf0[r, pl.ds(c * LANES, LANES)] = jnp.zeros(
                    (LANES,), jnp.float32)
            return carry
        lax.fori_loop(0, K, zrow, 0)

        zbase = sid * rpt
        nfull = rpt // K
        rem = rpt - nfull * K

        def zcp(i, carry):
            pltpu.sync_copy(buf0, acc.at[pl.ds(zbase + i * K, K)])
            return carry
        lax.fori_loop(0, nfull, zcp, 0)
        if rem:
            pltpu.sync_copy(buf0.at[pl.ds(0, rem)],
                            acc.at[pl.ds(zbase + nfull * K, rem)])

        for s in range(NRING - 1):
            pltpu.async_copy(pk_hbm.at[tid, s], idxr.at[s], si[s])
            pltpu.async_copy(pw_hbm.at[tid, s], ewr.at[s], si[s])
        pltpu.make_async_copy(pk_hbm.at[tid, 0], idxr.at[0], si[0]).wait()
        pltpu.make_async_copy(pw_hbm.at[tid, 0], ewr.at[0], si[0]).wait()
        pltpu.async_copy(x_hbm.at[idxr.at[0, 0]], bufs[0], sg[0])

        plsc.subcore_barrier()

        def scale(slot, buf):
            def rowblk(g, c2):
                wv = ewr[slot, pl.ds(g * LANES, LANES)]
                for u in range(LANES):
                    w = wv[u]
                    r = g * LANES + u
                    for c in range(D // LANES):
                        sl = pl.ds(c * LANES, LANES)
                        buf[r, sl] = buf[r, sl] * w
                return c2
            lax.fori_loop(0, K // LANES, rowblk, 0)

        def quad(i, carry):
            for uu in range(NRING):
                j = i * NRING + uu
                u = uu % 2
                un = (uu + 1) % 2
                sn = (uu + 1) % NRING
                sf = (uu + 3) % NRING

                @pl.when(j + 1 < NCH)
                def _():
                    @pl.when(j >= 1)
                    def _():
                        pltpu.make_async_copy(
                            bufs[un], acc.at[idxr.at[sf, 1]], ss[un]).wait()
                    jp = jnp.minimum(j + 1, NCH - 1)
                    pltpu.make_async_copy(
                        pk_hbm.at[tid, jp], idxr.at[sn], si[sn]).wait()
                    pltpu.make_async_copy(
                        pw_hbm.at[tid, jp], ewr.at[sn], si[sn]).wait()
                    pltpu.async_copy(
                        x_hbm.at[idxr.at[sn, 0]], bufs[un], sg[un])

                @pl.when(j + 3 < NCH)
                def _():
                    jf = jnp.minimum(j + 3, NCH - 1)
                    pltpu.async_copy(
                        pk_hbm.at[tid, jf], idxr.at[sf], si[sf])
                    pltpu.async_copy(
                        pw_hbm.at[tid, jf], ewr.at[sf], si[sf])

                pltpu.make_async_copy(
                    x_hbm.at[idxr.at[uu, 0]], bufs[u], sg[u]).wait()
                scale(uu, bufs[u])
                pltpu.async_copy(bufs[u], acc.at[idxr.at[uu, 1]], ss[u],
                                 add=True)
            return carry
        lax.fori_loop(0, NCH // NRING, quad, 0)

        for uu in range(NRING - 2, NRING):
            pltpu.make_async_copy(
                bufs[uu % 2], acc.at[idxr.at[uu, 1]], ss[uu % 2]).wait()

        plsc.subcore_barrier()
        pltpu.sync_copy(acc.at[pl.ds(zbase, rpt)],
                        out_hbm.at[cid, pl.ds(zbase, rpt)])

    return sc(x, pk, pw)


def _tc_combine_matmul(P, W, b, N):
    _, _, D = P.shape
    DO = W.shape[1]
    BM = 1000

    def body(p_ref, w_ref, b_ref, o_ref):
        s = p_ref[0] + p_ref[1]
        o_ref[...] = (
            jnp.dot(s, w_ref[...], preferred_element_type=jnp.float32)
            + b_ref[...]
        )

    return pl.pallas_call(
        body,
        grid=(N // BM,),
        in_specs=[
            pl.BlockSpec((NSC, BM, D), lambda i: (0, i, 0)),
            pl.BlockSpec((D, DO), lambda i: (0, 0)),
            pl.BlockSpec((1, DO), lambda i: (0, 0)),
        ],
        out_specs=pl.BlockSpec((BM, DO), lambda i: (i, 0)),
        out_shape=jax.ShapeDtypeStruct((N, DO), jnp.float32),
    )(P, W, b.reshape(1, DO))


def kernel(input, adj, edge_weight, W, b):
    x = input
    N, D = x.shape
    E = edge_weight.shape[0]

    ept = -(-E // NT)
    NCH = NRING * (-(-ept // (NRING * K)))
    EPT = NCH * K

    col = adj[1]
    row = adj[0]
    if E % NT:
        pad0 = NT * ept - E
        col = jnp.pad(col, (0, pad0))
        row = jnp.pad(row, (0, pad0))
        ew = jnp.pad(edge_weight, (0, pad0))
    else:
        ew = edge_weight
    colv = jnp.pad(col.reshape(NT, ept), ((0, 0), (0, EPT - ept)))
    rowv = jnp.pad(row.reshape(NT, ept), ((0, 0), (0, EPT - ept)))
    eww = jnp.pad(ew.reshape(NT, ept), ((0, 0), (0, EPT - ept)))

    pk = jnp.stack(
        [colv.reshape(NT, NCH, K), rowv.reshape(NT, NCH, K)], axis=2)
    pw = eww.reshape(NT, NCH, K)

    P = _sc_aggregate(x, pk, pw, N, D, NCH)
    return _tc_combine_matmul(P, W, b, N)
